# baseline (speedup 1.0000x reference)
import functools

import jax
import jax.numpy as jnp
from jax import lax
from jax.experimental import pallas as pl
from jax.experimental.pallas import tpu as pltpu
from jax.experimental.pallas import tpu_sc as plsc

_BINS = 32
_D = 128
_L = 512
_B = 2
_N = _B * _L * _L
_CHUNK = 128
_NQ = _L // _CHUNK


def _body(seq_hbm, emb_hbm, out_hbm, s_v, idx_v, rows_v, sem_g, sem_s):
    nc = 2
    wid = lax.axis_index("s") * nc + lax.axis_index("c")
    pairs_per_w = (_B * _L) // 32
    p0 = wid * pairs_per_w
    b = p0 // _L
    i0 = p0 % _L

    pltpu.sync_copy(seq_hbm.at[b], s_v.at[pl.ds(0, _L)])

    def pair_step(t, carry):
        i = i0 + t
        si = jnp.full((16,), s_v[pl.ds(i, 16)][0], jnp.int32)
        row0 = b * (_L * _L) + i * _L
        for q in range(_NQ):
            buf = q % 2
            for k in range(_CHUNK // 16):
                sj = s_v[pl.ds(q * _CHUNK + k * 16, 16)]
                d = jnp.clip(si - sj, -_BINS, _BINS) + (_BINS + 1)
                idx_v[buf, pl.ds(k * 16, 16)] = d
            pltpu.async_copy(
                emb_hbm.at[idx_v.at[buf]], rows_v.at[buf], sem_g
            ).wait()
            pltpu.async_copy(
                rows_v.at[buf], out_hbm.at[pl.ds(row0 + q * _CHUNK, _CHUNK)],
                sem_s,
            ).wait()
        return carry

    lax.fori_loop(0, pairs_per_w, pair_step, 0)


@jax.jit
def _run(seq_idx, embedding):
    mesh = plsc.VectorSubcoreMesh(core_axis_name="c", subcore_axis_name="s")
    f = functools.partial(
        pl.kernel,
        out_type=jax.ShapeDtypeStruct((_N, _D), jnp.float32),
        mesh=mesh,
        scratch_types=[
            pltpu.VMEM((_L + 16,), jnp.int32),
            pltpu.VMEM((2, _CHUNK), jnp.int32),
            pltpu.VMEM((2, _CHUNK, _D), jnp.float32),
            pltpu.SemaphoreType.DMA,
            pltpu.SemaphoreType.DMA,
        ],
    )(_body)
    out = f(seq_idx, embedding)
    return out.reshape(_B, _L, _L, _D)


def kernel(seq_idx, embedding):
    return _run(seq_idx, embedding)


# speedup vs baseline: 44.2026x; 44.2026x over previous
import functools

import jax
import jax.numpy as jnp
from jax import lax
from jax.experimental import pallas as pl
from jax.experimental.pallas import tpu as pltpu
from jax.experimental.pallas import tpu_sc as plsc

_BINS = 32
_D = 128
_L = 512
_B = 2
_V = 2 * _BINS + 2
_N = _B * _L * _L
_CHUNK = 128
_NQ = _L // _CHUNK


def _body(seq_hbm, emb_hbm, out_hbm, s_v, emb_v, idx_v, rows_v, sem_g, sem_s):
    nc = 2
    wid = lax.axis_index("s") * nc + lax.axis_index("c")
    pairs_per_w = (_B * _L) // 32
    p0 = wid * pairs_per_w
    b = p0 // _L
    i0 = p0 % _L

    pltpu.sync_copy(seq_hbm.at[b], s_v.at[pl.ds(0, _L)])

    @pl.when(lax.axis_index("s") == 0)
    def _stage():
        pltpu.sync_copy(emb_hbm, emb_v)

    plsc.subcore_barrier()

    def pair_step(t, carry):
        i = i0 + t
        si = jnp.full((16,), s_v[pl.ds(i, 16)][0], jnp.int32)
        row0 = b * (_L * _L) + i * _L
        for q in range(_NQ):
            for k in range(_CHUNK // 16):
                sj = s_v[pl.ds(q * _CHUNK + k * 16, 16)]
                d = jnp.clip(si - sj, -_BINS, _BINS) + (_BINS + 1)
                idx_v[q, pl.ds(k * 16, 16)] = d
            @pl.when(t > 0)
            def _drain():
                pltpu.make_async_copy(
                    rows_v.at[q], out_hbm.at[pl.ds(0, _CHUNK)], sem_s
                ).wait()

            pltpu.async_copy(
                emb_v.at[idx_v.at[q]], rows_v.at[q], sem_g
            ).wait()
            pltpu.async_copy(
                rows_v.at[q], out_hbm.at[pl.ds(row0 + q * _CHUNK, _CHUNK)],
                sem_s,
            )
        return carry

    lax.fori_loop(0, pairs_per_w, pair_step, 0)

    for q in range(_NQ):
        pltpu.make_async_copy(
            rows_v.at[q], out_hbm.at[pl.ds(0, _CHUNK)], sem_s
        ).wait()


@jax.jit
def _run(seq_idx, embedding):
    mesh = plsc.VectorSubcoreMesh(core_axis_name="c", subcore_axis_name="s")
    f = functools.partial(
        pl.kernel,
        out_type=jax.ShapeDtypeStruct((_N, _D), jnp.float32),
        mesh=mesh,
        scratch_types=[
            pltpu.VMEM((_L + 16,), jnp.int32),
            pltpu.VMEM_SHARED((_V, _D), jnp.float32),
            pltpu.VMEM((_NQ, _CHUNK), jnp.int32),
            pltpu.VMEM((_NQ, _CHUNK, _D), jnp.float32),
            pltpu.SemaphoreType.DMA,
            pltpu.SemaphoreType.DMA,
        ],
    )(_body)
    out = f(seq_idx, embedding)
    return out.reshape(_B, _L, _L, _D)


def kernel(seq_idx, embedding):
    return _run(seq_idx, embedding)
